# Initial kernel scaffold; baseline (speedup 1.0000x reference)
import jax
import jax.numpy as jnp
from jax.experimental import pallas as pl


def kernel(x, edge_index, batch, pre_lin_W, pre_lin_b, pre_W, pre_b, post_W, post_b, lin_W, lin_b, bn_gamma, bn_beta, mlp_W, mlp_b):
    raise NotImplementedError("write your pallas kernel here")


# speedup vs baseline: 50.7030x; 50.7030x over previous
import functools
import jax
import jax.numpy as jnp
from jax import lax
from jax.experimental import pallas as pl
from jax.experimental.pallas import tpu as pltpu
from jax.experimental.pallas import tpu_sc as plsc

N = 10000
E = 160000
G = 512
T = 5
F = 80
FO = 16
NL = 4

NP = 10240
FW = 512
FG = 4
NEG = 8
CH = 128
BLK = 256
NBLK = NP // BLK
TF = T * F

f32 = jnp.float32
i32 = jnp.int32
bf16 = jnp.bfloat16


def _prep_body(x_ref, olf_ref, ohf_ref, plw_ref, plb_ref, op_ref, aux_ref):
    deg = ohf_ref[...] - olf_ref[...]
    op_ref[...] = x_ref[:, 0:1] * plw_ref[...] + plb_ref[...]
    ald = jnp.sum(jnp.log(deg + 1.0)) / float(N)
    degc = jnp.maximum(deg, 1.0)
    ldc = jnp.log(degc + 1.0)
    amp = ldc / ald
    att = ald / ldc
    invc = 1.0 / degc
    has = (deg > 0.0).astype(f32)
    aux_ref[...] = jnp.concatenate(
        [deg, amp, att, invc, has, jnp.zeros((NP, 3), f32)], axis=1)


def _tc_prep(xpad, olf, ohf, plwT, plb):
    return pl.pallas_call(
        _prep_body,
        out_shape=(jax.ShapeDtypeStruct((NP, F), f32),
                   jax.ShapeDtypeStruct((NP, 8), f32)),
    )(xpad, olf, ohf, plwT, plb)


def _ab_body(first, y_ref, bnp_ref, wd_ref, ws_ref, pb_ref,
             a_ref, b0_ref, b1_ref, b2_ref, b3_ref):
    y = y_ref[...]
    if first:
        op = y
    else:
        op = jnp.maximum(y * bnp_ref[0:1, :] + bnp_ref[1:2, :], 0.0)
    opb = op.astype(bf16)
    a_ref[...] = jnp.dot(opb, wd_ref[...], preferred_element_type=f32) + pb_ref[...]
    bfull = jnp.dot(opb, ws_ref[...], preferred_element_type=f32)
    b0_ref[...] = bfull[:, 0:128]
    b1_ref[...] = bfull[:, 128:256]
    b2_ref[...] = bfull[:, 256:384]
    b3_ref[...] = bfull[:, 384:512]


def _tc_ab(first, y, bnp, wdT, wsT, pbp):
    bspec = lambda: pl.BlockSpec((BLK, 128), lambda i: (i, 0))
    return pl.pallas_call(
        functools.partial(_ab_body, first),
        grid=(NBLK,),
        in_specs=[pl.BlockSpec((BLK, F), lambda i: (i, 0)),
                  pl.BlockSpec((2, F), lambda i: (0, 0)),
                  pl.BlockSpec((F, FW), lambda i: (0, 0)),
                  pl.BlockSpec((F, FW), lambda i: (0, 0)),
                  pl.BlockSpec((1, FW), lambda i: (0, 0))],
        out_specs=(pl.BlockSpec((BLK, FW), lambda i: (i, 0)),
                   bspec(), bspec(), bspec(), bspec()),
        out_shape=(jax.ShapeDtypeStruct((NP, FW), f32),
                   jax.ShapeDtypeStruct((NP, 128), f32),
                   jax.ShapeDtypeStruct((NP, 128), f32),
                   jax.ShapeDtypeStruct((NP, 128), f32),
                   jax.ShapeDtypeStruct((NP, 128), f32)),
    )(y, bnp, wdT, wsT, pbp)


def _combine_body(first, y_ref, bnp_ref, a_ref, st_ref, aux_ref,
                  wx_ref, wa_ref, wb_ref, wc_ref, pb_ref, lw_ref, lb_ref,
                  ynew_ref, bn_ref, acc_ref):
    i = pl.program_id(0)
    y = y_ref[...]
    if first:
        op = y
    else:
        op = jnp.maximum(y * bnp_ref[0:1, :] + bnp_ref[1:2, :], 0.0)
    A = a_ref[...]
    st = st_ref[...]
    SB = jnp.concatenate([st[k, :, 0, :] for k in range(FG)], axis=1)
    SQ = jnp.concatenate([st[k, :, 1, :] for k in range(FG)], axis=1)
    MN = jnp.concatenate([st[k, :, 2, :] for k in range(FG)], axis=1)
    MX = jnp.concatenate([st[k, :, 3, :] for k in range(FG)], axis=1)
    aux = aux_ref[...]
    deg = aux[:, 0:1]
    amp = aux[:, 1:2]
    att = aux[:, 2:3]
    invc = aux[:, 3:4]
    has = aux[:, 4:5] > 0.0
    zero = jnp.zeros_like(A)
    mean = jnp.where(has, (deg * A + SB) * invc, zero)
    meanB = SB * invc
    var = jnp.where(has, SQ * invc - meanB * meanB, zero)
    std = jnp.sqrt(jnp.maximum(var, 0.0) + 1e-5)
    mn = jnp.where(has, A + MN, zero)
    mx = jnp.where(has, A + MX, zero)
    outs = []
    for t in range(T):
        sl = slice(t * F, (t + 1) * F)
        P = jnp.concatenate([mean[:, sl], mn[:, sl], mx[:, sl], std[:, sl]],
                            axis=1)
        o = (jnp.dot(op.astype(bf16), wx_ref[t], preferred_element_type=f32)
             + jnp.dot(P.astype(bf16), wa_ref[t], preferred_element_type=f32)
             + jnp.dot((amp * P).astype(bf16), wb_ref[t], preferred_element_type=f32)
             + jnp.dot((att * P).astype(bf16), wc_ref[t], preferred_element_type=f32)
             + pb_ref[0, t][None, :])
        outs.append(o)
    out80 = jnp.concatenate(outs, axis=1)
    y2 = jnp.dot(out80.astype(bf16), lw_ref[...], preferred_element_type=f32) + lb_ref[...]
    ynew_ref[...] = y2
    rid = i * BLK + lax.broadcasted_iota(i32, (BLK, 1), 0)
    m = (rid < N).astype(f32)
    ym = y2 * m

    @pl.when(i == 0)
    def _():
        acc_ref[...] = jnp.zeros_like(acc_ref)

    acc_ref[0:1, :] += jnp.sum(ym, axis=0, keepdims=True)
    acc_ref[1:2, :] += jnp.sum(ym * y2, axis=0, keepdims=True)

    @pl.when(i == NBLK - 1)
    def _():
        bn_ref[...] = acc_ref[...]


def _tc_combine(first, y, bnp, A, stats, aux, wxT, waT, wbT, wcT, pb, lwT, lb):
    return pl.pallas_call(
        functools.partial(_combine_body, first),
        grid=(NBLK,),
        in_specs=[pl.BlockSpec((BLK, F), lambda i: (i, 0)),
                  pl.BlockSpec((2, F), lambda i: (0, 0)),
                  pl.BlockSpec((BLK, FW), lambda i: (i, 0)),
                  pl.BlockSpec((FG, BLK, 4, 128), lambda i: (0, i, 0, 0)),
                  pl.BlockSpec((BLK, 8), lambda i: (i, 0)),
                  pl.BlockSpec((T, F, FO), lambda i: (0, 0, 0)),
                  pl.BlockSpec((T, 4 * F, FO), lambda i: (0, 0, 0)),
                  pl.BlockSpec((T, 4 * F, FO), lambda i: (0, 0, 0)),
                  pl.BlockSpec((T, 4 * F, FO), lambda i: (0, 0, 0)),
                  pl.BlockSpec((1, T, FO), lambda i: (0, 0, 0)),
                  pl.BlockSpec((F, F), lambda i: (0, 0)),
                  pl.BlockSpec((1, F), lambda i: (0, 0))],
        out_specs=(pl.BlockSpec((BLK, F), lambda i: (i, 0)),
                   pl.BlockSpec((2, F), lambda i: (0, 0))),
        out_shape=(jax.ShapeDtypeStruct((NP, F), f32),
                   jax.ShapeDtypeStruct((2, F), f32)),
        scratch_shapes=[pltpu.VMEM((2, F), f32)],
    )(y, bnp, A, stats, aux, wxT, waT, wbT, wcT, pb, lwT, lb)


def _bnparams_body(bn_ref, g_ref, b_ref, out_ref):
    s = bn_ref[0:1, :]
    q = bn_ref[1:2, :]
    mean = s / float(N)
    var = q / float(N) - mean * mean
    mult = g_ref[...] * lax.rsqrt(var + 1e-5)
    addr = b_ref[...] - mean * mult
    out_ref[...] = jnp.concatenate([mult, addr], axis=0)


def _tc_bnparams(bn, gamma, beta):
    return pl.pallas_call(
        _bnparams_body,
        out_shape=jax.ShapeDtypeStruct((2, F), f32),
    )(bn, gamma, beta)


def _t1_body(y_ref, bnp_ref, x_ref, lxc_ref):
    op = jnp.maximum(y_ref[...] * bnp_ref[0:1, :] + bnp_ref[1:2, :], 0.0)
    xc = op[:, 0:40] * x_ref[:, 1:2] + op[:, 40:80]
    l = jnp.log(xc + 1e-6)
    lxc_ref[...] = jnp.concatenate([l, jnp.zeros((BLK, 88), f32)], axis=1)


def _tc_t1(y4, bnp, xpad):
    return pl.pallas_call(
        _t1_body,
        grid=(NBLK,),
        in_specs=[pl.BlockSpec((BLK, F), lambda i: (i, 0)),
                  pl.BlockSpec((2, F), lambda i: (0, 0)),
                  pl.BlockSpec((BLK, 2), lambda i: (i, 0))],
        out_specs=pl.BlockSpec((BLK, 128), lambda i: (i, 0)),
        out_shape=jax.ShapeDtypeStruct((NP, 128), f32),
    )(y4, bnp, xpad)


def _t2_body(ls_ref, lxc_ref, aux_ref, score_ref):
    has = aux_ref[:, 4:5] > 0.0
    ls = jnp.where(has, ls_ref[...], jnp.zeros_like(ls_ref))
    sc = jnp.exp(ls + lxc_ref[:, 0:48])
    score_ref[...] = jnp.concatenate([sc, jnp.zeros((BLK, 80), f32)], axis=1)


def _tc_t2(LS, lxc, aux):
    return pl.pallas_call(
        _t2_body,
        grid=(NBLK,),
        in_specs=[pl.BlockSpec((BLK, 48), lambda i: (i, 0)),
                  pl.BlockSpec((BLK, 128), lambda i: (i, 0)),
                  pl.BlockSpec((BLK, 8), lambda i: (i, 0))],
        out_specs=pl.BlockSpec((BLK, 128), lambda i: (i, 0)),
        out_shape=jax.ShapeDtypeStruct((NP, 128), f32),
    )(LS, lxc, aux)


def _t3_body(gs_ref, blf_ref, bhf_ref, w_ref, b_ref, out_ref):
    cnt = bhf_ref[...] - blf_ref[...]
    gs = jnp.where(cnt > 0.0, gs_ref[:, 0:40], jnp.zeros((G, 40), f32))
    pooled = gs * (1.0 / jnp.maximum(cnt, 1.0))
    out_ref[...] = jnp.dot(pooled.astype(bf16), w_ref[...], preferred_element_type=f32) + b_ref[...]


def _tc_t3(gsum, blf, bhf, mlpWT, mlpb):
    return pl.pallas_call(
        _t3_body,
        out_shape=jax.ShapeDtypeStruct((G, 1), f32),
    )(gsum, blf, bhf, mlpWT, mlpb)


_MESH_CACHE = []


def _mesh():
    if not _MESH_CACHE:
        _MESH_CACHE.append(
            plsc.VectorSubcoreMesh(core_axis_name="c", subcore_axis_name="s"))
    return _MESH_CACHE[0]


def _sc_stats_kernel(b_hbm, srcs_hbm, dsts_hbm, offs_hbm, out_hbm,
                     idx_v, rows_v, dst_sm, off_sm, stage_v, sems, gsem):
    cid = lax.axis_index("c")
    sid = lax.axis_index("s")
    wid = sid * 2 + cid
    eg = wid // FG
    fg = wid % FG
    lo = eg * (NP // NEG)
    hi = lo + (NP // NEG)
    pltpu.sync_copy(offs_hbm.at[pl.ds(lo, 8)], off_sm.at[pl.ds(0, 8)])
    pltpu.sync_copy(offs_hbm.at[pl.ds(hi, 8)], off_sm.at[pl.ds(8, 8)])
    ovec = off_sm[pl.ds(0, 16)]
    e0 = ovec[0]
    e1 = ovec[8]
    c0 = e0 // CH
    c1 = (e1 + CH - 1) // CH

    NVR = 128 // 16

    init_accs = ([jnp.zeros((1, 16), f32) for _ in range(2 * NVR)]
                 + [jnp.full((1, 16), jnp.inf, f32) for _ in range(NVR)]
                 + [jnp.full((1, 16), -jnp.inf, f32) for _ in range(NVR)])

    def edge_step(i, d, carry):
        cur = carry[0]
        fc = carry[1]
        accs = carry[2:]
        valid = jnp.logical_and(d >= lo, d < hi)
        dn = jnp.where(valid, d, -1)
        change = dn != cur

        def do_flush(ops):
            cur_, fc_, accs_ = ops[0], ops[1], ops[2:]
            slot = lax.rem(fc_, 4)

            @pl.when(cur_ >= 0)
            def _():
                @pl.when(fc_ >= 4)
                def _():
                    pltpu.make_async_copy(stage_v.at[pl.ds(slot * 4, 4)],
                                          out_hbm.at[0, 0],
                                          sems.at[slot]).wait()
                for k in range(NVR):
                    stage_v[pl.ds(slot * 4, 1), pl.ds(k * 16, 16)] = accs_[k]
                    stage_v[pl.ds(slot * 4 + 1, 1), pl.ds(k * 16, 16)] = accs_[NVR + k]
                    stage_v[pl.ds(slot * 4 + 2, 1), pl.ds(k * 16, 16)] = accs_[2 * NVR + k]
                    stage_v[pl.ds(slot * 4 + 3, 1), pl.ds(k * 16, 16)] = accs_[3 * NVR + k]
                pltpu.async_copy(stage_v.at[pl.ds(slot * 4, 4)],
                                 out_hbm.at[fg, cur_], sems.at[slot])
            newfc = jnp.where(cur_ >= 0, fc_ + 1, fc_)
            return (dn, newfc) + tuple(init_accs)

        def no_flush(ops):
            return ops

        carry2 = lax.cond(change, do_flush, no_flush, (cur, fc) + tuple(accs))
        cur2 = carry2[0]
        fc2 = carry2[1]
        accs2 = list(carry2[2:])
        new = []
        for k in range(NVR):
            b = rows_v[pl.ds(i, 1), pl.ds(k * 16, 16)]
            new.append(accs2[k] + b)
            accs2[NVR + k] = accs2[NVR + k] + b * b
            accs2[2 * NVR + k] = jnp.minimum(accs2[2 * NVR + k], b)
            accs2[3 * NVR + k] = jnp.maximum(accs2[3 * NVR + k], b)
        for k in range(NVR):
            accs2[k] = new[k]
        return (cur2, fc2) + tuple(accs2)

    def group_step(g, carry):
        dvec = dst_sm[pl.ds(g * 16, 16)]
        for j in range(16):
            carry = edge_step(g * 16 + j, dvec[j], carry)
        return carry

    def chunk_body(ci, carry):
        pltpu.sync_copy(srcs_hbm.at[fg, pl.ds(ci * CH, CH)], idx_v)
        pltpu.sync_copy(dsts_hbm.at[pl.ds(ci * CH, CH)], dst_sm)
        pltpu.async_copy(b_hbm.at[idx_v], rows_v, gsem).wait()
        return lax.fori_loop(0, CH // 16, group_step, carry)

    carry0 = (jnp.int32(-1), jnp.int32(0)) + tuple(init_accs)
    carry = lax.fori_loop(c0, c1, chunk_body, carry0)
    cur = carry[0]
    fc = carry[1]
    accs = carry[2:]

    @pl.when(cur >= 0)
    def _():
        slot = lax.rem(fc, 4)

        @pl.when(fc >= 4)
        def _():
            pltpu.make_async_copy(stage_v.at[pl.ds(slot * 4, 4)],
                                  out_hbm.at[0, 0], sems.at[slot]).wait()
        for k in range(8):
            stage_v[pl.ds(slot * 4, 1), pl.ds(k * 16, 16)] = accs[k]
            stage_v[pl.ds(slot * 4 + 1, 1), pl.ds(k * 16, 16)] = accs[8 + k]
            stage_v[pl.ds(slot * 4 + 2, 1), pl.ds(k * 16, 16)] = accs[16 + k]
            stage_v[pl.ds(slot * 4 + 3, 1), pl.ds(k * 16, 16)] = accs[24 + k]
        pltpu.async_copy(stage_v.at[pl.ds(slot * 4, 4)], out_hbm.at[fg, cur],
                         sems.at[slot])

    fcf = jnp.where(cur >= 0, fc + 1, fc)
    for k in range(4):
        @pl.when(fcf > k)
        def _(k=k):
            pltpu.make_async_copy(stage_v.at[pl.ds(k * 4, 4)],
                                  out_hbm.at[0, 0], sems.at[k]).wait()


def _sc_stats(Bst, srcs4, dsts, offs):
    kern = pl.kernel(
        _sc_stats_kernel,
        out_type=jax.ShapeDtypeStruct((FG, NP, 4, 128), f32),
        mesh=_mesh(),
        scratch_types=[
            pltpu.VMEM((CH,), i32),
            pltpu.VMEM((CH, 128), f32),
            pltpu.VMEM((CH,), i32),
            pltpu.VMEM((16,), i32),
            pltpu.VMEM((16, 128), f32),
            pltpu.SemaphoreType.DMA((4,)),
            pltpu.SemaphoreType.DMA,
        ],
    )
    return kern(Bst, srcs4, dsts, offs)


def _sc_segsum_kernel(nseg, width, data_hbm, idx_hbm, keys_hbm, offs_hbm,
                      out_hbm, idx_v, rows_v, key_sm, off_sm, stage_v, sems,
                      gsem, linear):
    cid = lax.axis_index("c")
    sid = lax.axis_index("s")
    wid = sid * 2 + cid
    lo = wid * (nseg // 32)
    hi = lo + (nseg // 32)
    pltpu.sync_copy(offs_hbm.at[pl.ds(lo, 8)], off_sm.at[pl.ds(0, 8)])
    pltpu.sync_copy(offs_hbm.at[pl.ds(hi, 8)], off_sm.at[pl.ds(8, 8)])
    ovec = off_sm[pl.ds(0, 16)]
    e0 = ovec[0]
    e1 = ovec[8]
    c0 = e0 // CH
    c1 = (e1 + CH - 1) // CH
    NVR = width // 16

    init_accs = [jnp.zeros((1, 16), f32) for _ in range(NVR)]

    def edge_step(i, d, carry):
        cur = carry[0]
        fc = carry[1]
        accs = carry[2:]
        valid = jnp.logical_and(d >= lo, d < hi)
        dn = jnp.where(valid, d, -1)
        change = dn != cur

        def do_flush(ops):
            cur_, fc_, accs_ = ops[0], ops[1], ops[2:]
            slot = lax.rem(fc_, 4)

            @pl.when(cur_ >= 0)
            def _():
                @pl.when(fc_ >= 4)
                def _():
                    pltpu.make_async_copy(stage_v.at[pl.ds(slot, 1)],
                                          out_hbm.at[pl.ds(0, 1)],
                                          sems.at[slot]).wait()
                for k in range(NVR):
                    stage_v[pl.ds(slot, 1), pl.ds(k * 16, 16)] = accs_[k]
                pltpu.async_copy(stage_v.at[pl.ds(slot, 1)],
                                 out_hbm.at[pl.ds(cur_, 1)], sems.at[slot])
            newfc = jnp.where(cur_ >= 0, fc_ + 1, fc_)
            return (dn, newfc) + tuple(init_accs)

        def no_flush(ops):
            return ops

        carry2 = lax.cond(change, do_flush, no_flush, (cur, fc) + tuple(accs))
        cur2 = carry2[0]
        fc2 = carry2[1]
        accs2 = list(carry2[2:])
        for k in range(NVR):
            accs2[k] = accs2[k] + rows_v[pl.ds(i, 1), pl.ds(k * 16, 16)]
        return (cur2, fc2) + tuple(accs2)

    def group_step(g, carry):
        dvec = key_sm[pl.ds(g * 16, 16)]
        for j in range(16):
            carry = edge_step(g * 16 + j, dvec[j], carry)
        return carry

    def chunk_body(ci, carry):
        pltpu.sync_copy(keys_hbm.at[pl.ds(ci * CH, CH)], key_sm)
        if linear:
            pltpu.sync_copy(data_hbm.at[pl.ds(ci * CH, CH)], rows_v)
        else:
            pltpu.sync_copy(idx_hbm.at[pl.ds(ci * CH, CH)], idx_v)
            pltpu.async_copy(data_hbm.at[idx_v], rows_v, gsem).wait()
        return lax.fori_loop(0, CH // 16, group_step, carry)

    carry0 = (jnp.int32(-1), jnp.int32(0)) + tuple(init_accs)
    carry = lax.fori_loop(c0, c1, chunk_body, carry0)
    cur = carry[0]
    fc = carry[1]
    accs = carry[2:]

    @pl.when(cur >= 0)
    def _():
        slot = lax.rem(fc, 4)

        @pl.when(fc >= 4)
        def _():
            pltpu.make_async_copy(stage_v.at[pl.ds(slot, 1)],
                                  out_hbm.at[pl.ds(0, 1)], sems.at[slot]).wait()
        for k in range(NVR):
            stage_v[pl.ds(slot, 1), pl.ds(k * 16, 16)] = accs[k]
        pltpu.async_copy(stage_v.at[pl.ds(slot, 1)], out_hbm.at[pl.ds(cur, 1)],
                         sems.at[slot])

    fcf = jnp.where(cur >= 0, fc + 1, fc)
    for k in range(4):
        @pl.when(fcf > k)
        def _(k=k):
            pltpu.make_async_copy(stage_v.at[pl.ds(k, 1)],
                                  out_hbm.at[pl.ds(0, 1)], sems.at[k]).wait()


def _sc_edge_logsum(lxc, src_s, dst_s, offs):
    kern = pl.kernel(
        functools.partial(_sc_segsum_kernel, NP, 48, linear=False),
        out_type=jax.ShapeDtypeStruct((NP, 48), f32),
        mesh=_mesh(),
        scratch_types=[
            pltpu.VMEM((CH,), i32),
            pltpu.VMEM((CH, 128), f32),
            pltpu.VMEM((CH,), i32),
            pltpu.VMEM((16,), i32),
            pltpu.VMEM((4, 48), f32),
            pltpu.SemaphoreType.DMA((4,)),
            pltpu.SemaphoreType.DMA,
        ],
    )
    return kern(lxc, src_s, dst_s, offs)


def _sc_pool(score, batch_pad, boffs):
    kern = pl.kernel(
        functools.partial(_sc_segsum_kernel, G, 48, linear=True),
        out_type=jax.ShapeDtypeStruct((G, 48), f32),
        mesh=_mesh(),
        scratch_types=[
            pltpu.VMEM((CH,), i32),
            pltpu.VMEM((CH, 128), f32),
            pltpu.VMEM((CH,), i32),
            pltpu.VMEM((16,), i32),
            pltpu.VMEM((4, 48), f32),
            pltpu.SemaphoreType.DMA((4,)),
            pltpu.SemaphoreType.DMA,
        ],
    )
    return kern(score, score, batch_pad, boffs)


def kernel(x, edge_index, batch, pre_lin_W, pre_lin_b, pre_W, pre_b,
           post_W, post_b, lin_W, lin_b, bn_gamma, bn_beta, mlp_W, mlp_b):
    src = edge_index[0]
    dst = edge_index[1]

    perm = jnp.argsort(dst)
    src_s = src[perm].astype(i32)
    dst_s = dst[perm].astype(i32)
    offs = jnp.searchsorted(dst_s, jnp.arange(NP + 16, dtype=i32),
                            side='left').astype(i32)
    srcs4 = src_s[None, :] + (jnp.arange(FG, dtype=i32) * NP)[:, None]
    boffs = jnp.searchsorted(batch.astype(i32), jnp.arange(528, dtype=i32),
                             side='left').astype(i32)
    batch_pad = jnp.pad(batch.astype(i32), (0, NP - N), constant_values=G + 7)

    xpad = jnp.pad(x, ((0, NP - N), (0, 0)))
    olf = offs[:NP].astype(f32)[:, None]
    ohf = offs[1:NP + 1].astype(f32)[:, None]
    blf = boffs[:G].astype(f32)[:, None]
    bhf = boffs[1:G + 1].astype(f32)[:, None]

    wflat = pre_W.reshape(NL, TF, 2 * F)
    wdT = jnp.pad(wflat[:, :, :F].transpose(0, 2, 1),
                  ((0, 0), (0, 0), (0, FW - TF))).astype(bf16)
    wsT = jnp.pad(wflat[:, :, F:].transpose(0, 2, 1),
                  ((0, 0), (0, 0), (0, FW - TF))).astype(bf16)
    pbp = jnp.pad(pre_b.reshape(NL, 1, TF), ((0, 0), (0, 0), (0, FW - TF)))
    wxT = post_W[:, :, :, :F].transpose(0, 1, 3, 2).astype(bf16)
    waT = post_W[:, :, :, F:5 * F].transpose(0, 1, 3, 2).astype(bf16)
    wbT = post_W[:, :, :, 5 * F:9 * F].transpose(0, 1, 3, 2).astype(bf16)
    wcT = post_W[:, :, :, 9 * F:].transpose(0, 1, 3, 2).astype(bf16)
    pbl = post_b.reshape(NL, 1, T, FO)
    lwT = lin_W.transpose(0, 2, 1).astype(bf16)
    lbl = lin_b.reshape(NL, 1, F)

    op0, aux = _tc_prep(xpad, olf, ohf, pre_lin_W.T, pre_lin_b.reshape(1, F))

    y = op0
    bnp = jnp.zeros((2, F), f32)
    for l in range(NL):
        first = (l == 0)
        A, B0, B1, B2, B3 = _tc_ab(first, y, bnp, wdT[l], wsT[l], pbp[l])
        Bst = jnp.concatenate([B0, B1, B2, B3], axis=0)
        stats = _sc_stats(Bst, srcs4, dst_s, offs)
        y2, bn = _tc_combine(first, y, bnp, A, stats, aux,
                             wxT[l], waT[l], wbT[l], wcT[l], pbl[l],
                             lwT[l], lbl[l])
        bnp = _tc_bnparams(bn, bn_gamma[l].reshape(1, F), bn_beta[l].reshape(1, F))
        y = y2

    lxc = _tc_t1(y, bnp, xpad)
    LS = _sc_edge_logsum(lxc, src_s, dst_s, offs)
    score = _tc_t2(LS, lxc, aux)
    gsum = _sc_pool(score, batch_pad, boffs)
    out = _tc_t3(gsum, blf, bhf, mlp_W.T.astype(bf16), mlp_b.reshape(1, 1))
    return out


# speedup vs baseline: 59.5520x; 1.1745x over previous
import functools
import jax
import jax.numpy as jnp
from jax import lax
from jax.experimental import pallas as pl
from jax.experimental.pallas import tpu as pltpu
from jax.experimental.pallas import tpu_sc as plsc

N = 10000
E = 160000
G = 512
T = 5
F = 80
FO = 16
NL = 4

NP = 10240
FW = 512
FG = 4
NEG = 8
CH = 128
BLK = 256
NBLK = NP // BLK
TF = T * F

f32 = jnp.float32
i32 = jnp.int32
bf16 = jnp.bfloat16


def _prep_body(x_ref, olf_ref, ohf_ref, plw_ref, plb_ref, op_ref, aux_ref):
    deg = ohf_ref[...] - olf_ref[...]
    op_ref[...] = x_ref[:, 0:1] * plw_ref[...] + plb_ref[...]
    ald = jnp.sum(jnp.log(deg + 1.0)) / float(N)
    degc = jnp.maximum(deg, 1.0)
    ldc = jnp.log(degc + 1.0)
    amp = ldc / ald
    att = ald / ldc
    invc = 1.0 / degc
    has = (deg > 0.0).astype(f32)
    aux_ref[...] = jnp.concatenate(
        [deg, amp, att, invc, has, jnp.zeros((NP, 3), f32)], axis=1)


def _tc_prep(xpad, olf, ohf, plwT, plb):
    return pl.pallas_call(
        _prep_body,
        out_shape=(jax.ShapeDtypeStruct((NP, F), f32),
                   jax.ShapeDtypeStruct((NP, 8), f32)),
    )(xpad, olf, ohf, plwT, plb)


def _ab_body(first, y_ref, bnp_ref, wd_ref, ws_ref, pb_ref,
             a_ref, b0_ref, b1_ref, b2_ref, b3_ref):
    y = y_ref[...]
    if first:
        op = y
    else:
        op = jnp.maximum(y * bnp_ref[0:1, :] + bnp_ref[1:2, :], 0.0)
    opb = op.astype(bf16)
    a_ref[...] = jnp.dot(opb, wd_ref[...], preferred_element_type=f32) + pb_ref[...]
    bfull = jnp.dot(opb, ws_ref[...], preferred_element_type=f32)
    b0_ref[...] = bfull[:, 0:128]
    b1_ref[...] = bfull[:, 128:256]
    b2_ref[...] = bfull[:, 256:384]
    b3_ref[...] = bfull[:, 384:512]


def _tc_ab(first, y, bnp, wdT, wsT, pbp):
    bspec = lambda: pl.BlockSpec((BLK, 128), lambda i: (i, 0))
    return pl.pallas_call(
        functools.partial(_ab_body, first),
        grid=(NBLK,),
        in_specs=[pl.BlockSpec((BLK, F), lambda i: (i, 0)),
                  pl.BlockSpec((2, F), lambda i: (0, 0)),
                  pl.BlockSpec((F, FW), lambda i: (0, 0)),
                  pl.BlockSpec((F, FW), lambda i: (0, 0)),
                  pl.BlockSpec((1, FW), lambda i: (0, 0))],
        out_specs=(pl.BlockSpec((BLK, FW), lambda i: (i, 0)),
                   bspec(), bspec(), bspec(), bspec()),
        out_shape=(jax.ShapeDtypeStruct((NP, FW), f32),
                   jax.ShapeDtypeStruct((NP, 128), f32),
                   jax.ShapeDtypeStruct((NP, 128), f32),
                   jax.ShapeDtypeStruct((NP, 128), f32),
                   jax.ShapeDtypeStruct((NP, 128), f32)),
    )(y, bnp, wdT, wsT, pbp)


def _combine_body(first, y_ref, bnp_ref, a_ref, st_ref, aux_ref,
                  wx_ref, wa_ref, wb_ref, wc_ref, pb_ref, lw_ref, lb_ref,
                  ynew_ref, bn_ref, acc_ref):
    i = pl.program_id(0)
    y = y_ref[...]
    if first:
        op = y
    else:
        op = jnp.maximum(y * bnp_ref[0:1, :] + bnp_ref[1:2, :], 0.0)
    A = a_ref[...]
    st = st_ref[...]
    SB = jnp.concatenate([st[k, :, 0, :] for k in range(FG)], axis=1)
    SQ = jnp.concatenate([st[k, :, 1, :] for k in range(FG)], axis=1)
    MN = jnp.concatenate([st[k, :, 2, :] for k in range(FG)], axis=1)
    MX = jnp.concatenate([st[k, :, 3, :] for k in range(FG)], axis=1)
    aux = aux_ref[...]
    deg = aux[:, 0:1]
    amp = aux[:, 1:2]
    att = aux[:, 2:3]
    invc = aux[:, 3:4]
    has = aux[:, 4:5] > 0.0
    zero = jnp.zeros_like(A)
    mean = jnp.where(has, (deg * A + SB) * invc, zero)
    meanB = SB * invc
    var = jnp.where(has, SQ * invc - meanB * meanB, zero)
    std = jnp.sqrt(jnp.maximum(var, 0.0) + 1e-5)
    mn = jnp.where(has, A + MN, zero)
    mx = jnp.where(has, A + MX, zero)
    outs = []
    for t in range(T):
        sl = slice(t * F, (t + 1) * F)
        P = jnp.concatenate([mean[:, sl], mn[:, sl], mx[:, sl], std[:, sl]],
                            axis=1)
        o = (jnp.dot(op.astype(bf16), wx_ref[t], preferred_element_type=f32)
             + jnp.dot(P.astype(bf16), wa_ref[t], preferred_element_type=f32)
             + jnp.dot((amp * P).astype(bf16), wb_ref[t], preferred_element_type=f32)
             + jnp.dot((att * P).astype(bf16), wc_ref[t], preferred_element_type=f32)
             + pb_ref[0, t][None, :])
        outs.append(o)
    out80 = jnp.concatenate(outs, axis=1)
    y2 = jnp.dot(out80.astype(bf16), lw_ref[...], preferred_element_type=f32) + lb_ref[...]
    ynew_ref[...] = y2
    rid = i * BLK + lax.broadcasted_iota(i32, (BLK, 1), 0)
    m = (rid < N).astype(f32)
    ym = y2 * m

    @pl.when(i == 0)
    def _():
        acc_ref[...] = jnp.zeros_like(acc_ref)

    acc_ref[0:1, :] += jnp.sum(ym, axis=0, keepdims=True)
    acc_ref[1:2, :] += jnp.sum(ym * y2, axis=0, keepdims=True)

    @pl.when(i == NBLK - 1)
    def _():
        bn_ref[...] = acc_ref[...]


def _tc_combine(first, y, bnp, A, stats, aux, wxT, waT, wbT, wcT, pb, lwT, lb):
    return pl.pallas_call(
        functools.partial(_combine_body, first),
        grid=(NBLK,),
        in_specs=[pl.BlockSpec((BLK, F), lambda i: (i, 0)),
                  pl.BlockSpec((2, F), lambda i: (0, 0)),
                  pl.BlockSpec((BLK, FW), lambda i: (i, 0)),
                  pl.BlockSpec((FG, BLK, 4, 128), lambda i: (0, i, 0, 0)),
                  pl.BlockSpec((BLK, 8), lambda i: (i, 0)),
                  pl.BlockSpec((T, F, FO), lambda i: (0, 0, 0)),
                  pl.BlockSpec((T, 4 * F, FO), lambda i: (0, 0, 0)),
                  pl.BlockSpec((T, 4 * F, FO), lambda i: (0, 0, 0)),
                  pl.BlockSpec((T, 4 * F, FO), lambda i: (0, 0, 0)),
                  pl.BlockSpec((1, T, FO), lambda i: (0, 0, 0)),
                  pl.BlockSpec((F, F), lambda i: (0, 0)),
                  pl.BlockSpec((1, F), lambda i: (0, 0))],
        out_specs=(pl.BlockSpec((BLK, F), lambda i: (i, 0)),
                   pl.BlockSpec((2, F), lambda i: (0, 0))),
        out_shape=(jax.ShapeDtypeStruct((NP, F), f32),
                   jax.ShapeDtypeStruct((2, F), f32)),
        scratch_shapes=[pltpu.VMEM((2, F), f32)],
    )(y, bnp, A, stats, aux, wxT, waT, wbT, wcT, pb, lwT, lb)


def _bnparams_body(bn_ref, g_ref, b_ref, out_ref):
    s = bn_ref[0:1, :]
    q = bn_ref[1:2, :]
    mean = s / float(N)
    var = q / float(N) - mean * mean
    mult = g_ref[...] * lax.rsqrt(var + 1e-5)
    addr = b_ref[...] - mean * mult
    out_ref[...] = jnp.concatenate([mult, addr], axis=0)


def _tc_bnparams(bn, gamma, beta):
    return pl.pallas_call(
        _bnparams_body,
        out_shape=jax.ShapeDtypeStruct((2, F), f32),
    )(bn, gamma, beta)


def _t1_body(y_ref, bnp_ref, x_ref, lxc_ref):
    op = jnp.maximum(y_ref[...] * bnp_ref[0:1, :] + bnp_ref[1:2, :], 0.0)
    xc = op[:, 0:40] * x_ref[:, 1:2] + op[:, 40:80]
    l = jnp.log(xc + 1e-6)
    lxc_ref[...] = jnp.concatenate([l, jnp.zeros((BLK, 88), f32)], axis=1)


def _tc_t1(y4, bnp, xpad):
    return pl.pallas_call(
        _t1_body,
        grid=(NBLK,),
        in_specs=[pl.BlockSpec((BLK, F), lambda i: (i, 0)),
                  pl.BlockSpec((2, F), lambda i: (0, 0)),
                  pl.BlockSpec((BLK, 2), lambda i: (i, 0))],
        out_specs=pl.BlockSpec((BLK, 128), lambda i: (i, 0)),
        out_shape=jax.ShapeDtypeStruct((NP, 128), f32),
    )(y4, bnp, xpad)


def _t2_body(ls_ref, lxc_ref, aux_ref, score_ref):
    has = aux_ref[:, 4:5] > 0.0
    ls = jnp.where(has, ls_ref[...], jnp.zeros_like(ls_ref))
    sc = jnp.exp(ls + lxc_ref[:, 0:48])
    score_ref[...] = jnp.concatenate([sc, jnp.zeros((BLK, 80), f32)], axis=1)


def _tc_t2(LS, lxc, aux):
    return pl.pallas_call(
        _t2_body,
        grid=(NBLK,),
        in_specs=[pl.BlockSpec((BLK, 48), lambda i: (i, 0)),
                  pl.BlockSpec((BLK, 128), lambda i: (i, 0)),
                  pl.BlockSpec((BLK, 8), lambda i: (i, 0))],
        out_specs=pl.BlockSpec((BLK, 128), lambda i: (i, 0)),
        out_shape=jax.ShapeDtypeStruct((NP, 128), f32),
    )(LS, lxc, aux)


def _t3_body(gs_ref, blf_ref, bhf_ref, w_ref, b_ref, out_ref):
    cnt = bhf_ref[...] - blf_ref[...]
    gs = jnp.where(cnt > 0.0, gs_ref[:, 0:40], jnp.zeros((G, 40), f32))
    pooled = gs * (1.0 / jnp.maximum(cnt, 1.0))
    out_ref[...] = jnp.dot(pooled.astype(bf16), w_ref[...], preferred_element_type=f32) + b_ref[...]


def _tc_t3(gsum, blf, bhf, mlpWT, mlpb):
    return pl.pallas_call(
        _t3_body,
        out_shape=jax.ShapeDtypeStruct((G, 1), f32),
    )(gsum, blf, bhf, mlpWT, mlpb)


_MESH_CACHE = []


def _mesh():
    if not _MESH_CACHE:
        _MESH_CACHE.append(
            plsc.VectorSubcoreMesh(core_axis_name="c", subcore_axis_name="s"))
    return _MESH_CACHE[0]


def _sc_stats_kernel(b_hbm, srcs_hbm, dsts_hbm, offs_hbm, out_hbm,
                     idx_v, rows_v, dst_sm, off_sm, stage_v, sems, gsem):
    cid = lax.axis_index("c")
    sid = lax.axis_index("s")
    wid = sid * 2 + cid
    eg = wid // FG
    fg = wid % FG
    lo = eg * (NP // NEG)
    hi = lo + (NP // NEG)
    pltpu.sync_copy(offs_hbm.at[pl.ds(lo, 8)], off_sm.at[pl.ds(0, 8)])
    pltpu.sync_copy(offs_hbm.at[pl.ds(hi, 8)], off_sm.at[pl.ds(8, 8)])
    ovec = off_sm[pl.ds(0, 16)]
    e0 = ovec[0]
    e1 = ovec[8]
    c0 = e0 // CH
    c1 = (e1 + CH - 1) // CH

    NVR = 128 // 16

    init_accs = ([jnp.zeros((1, 16), f32) for _ in range(2 * NVR)]
                 + [jnp.full((1, 16), jnp.inf, f32) for _ in range(NVR)]
                 + [jnp.full((1, 16), -jnp.inf, f32) for _ in range(NVR)])

    def edge_step(i, d, carry):
        cur = carry[0]
        fc = carry[1]
        accs = carry[2:]
        valid = jnp.logical_and(d >= lo, d < hi)
        dn = jnp.where(valid, d, -1)
        change = dn != cur

        def do_flush(ops):
            cur_, fc_, accs_ = ops[0], ops[1], ops[2:]
            slot = lax.rem(fc_, 4)

            @pl.when(cur_ >= 0)
            def _():
                @pl.when(fc_ >= 4)
                def _():
                    pltpu.make_async_copy(stage_v.at[pl.ds(slot * 4, 4)],
                                          out_hbm.at[0, 0],
                                          sems.at[slot]).wait()
                for k in range(NVR):
                    stage_v[pl.ds(slot * 4, 1), pl.ds(k * 16, 16)] = accs_[k]
                    stage_v[pl.ds(slot * 4 + 1, 1), pl.ds(k * 16, 16)] = accs_[NVR + k]
                    stage_v[pl.ds(slot * 4 + 2, 1), pl.ds(k * 16, 16)] = accs_[2 * NVR + k]
                    stage_v[pl.ds(slot * 4 + 3, 1), pl.ds(k * 16, 16)] = accs_[3 * NVR + k]
                pltpu.async_copy(stage_v.at[pl.ds(slot * 4, 4)],
                                 out_hbm.at[fg, cur_], sems.at[slot])
            newfc = jnp.where(cur_ >= 0, fc_ + 1, fc_)
            return (dn, newfc) + tuple(init_accs)

        def no_flush(ops):
            return ops

        carry2 = lax.cond(change, do_flush, no_flush, (cur, fc) + tuple(accs))
        cur2 = carry2[0]
        fc2 = carry2[1]
        accs2 = list(carry2[2:])
        new = []
        for k in range(NVR):
            b = rows_v[pl.ds(i, 1), pl.ds(k * 16, 16)]
            new.append(accs2[k] + b)
            accs2[NVR + k] = accs2[NVR + k] + b * b
            accs2[2 * NVR + k] = jnp.minimum(accs2[2 * NVR + k], b)
            accs2[3 * NVR + k] = jnp.maximum(accs2[3 * NVR + k], b)
        for k in range(NVR):
            accs2[k] = new[k]
        return (cur2, fc2) + tuple(accs2)

    def group_step(g, carry):
        p = carry[-1]
        dvec = dst_sm[pl.ds(p * CH + g * 16, 16)]
        for j in range(16):
            carry = edge_step(p * CH + g * 16 + j, dvec[j], carry[:-1]) + (p,)
        return carry

    def prefetch(ci, p):
        pltpu.sync_copy(srcs_hbm.at[fg, pl.ds(ci * CH, CH)],
                        idx_v.at[pl.ds(p * CH, CH)])
        pltpu.sync_copy(dsts_hbm.at[pl.ds(ci * CH, CH)],
                        dst_sm.at[pl.ds(p * CH, CH)])
        pltpu.async_copy(b_hbm.at[idx_v.at[pl.ds(p * CH, CH)]],
                         rows_v.at[pl.ds(p * CH, CH)], gsem.at[p])

    @pl.when(c0 < c1)
    def _():
        prefetch(c0, 0)

    def chunk_body(ci, carry):
        p = lax.rem(ci - c0, 2)

        @pl.when(ci + 1 < c1)
        def _():
            prefetch(ci + 1, 1 - p)
        pltpu.make_async_copy(b_hbm.at[idx_v.at[pl.ds(p * CH, CH)]],
                              rows_v.at[pl.ds(p * CH, CH)], gsem.at[p]).wait()
        return lax.fori_loop(0, CH // 16, group_step, carry + (p,))[:-1]

    carry0 = (jnp.int32(-1), jnp.int32(0)) + tuple(init_accs)
    carry = lax.fori_loop(c0, c1, chunk_body, carry0)
    cur = carry[0]
    fc = carry[1]
    accs = carry[2:]

    @pl.when(cur >= 0)
    def _():
        slot = lax.rem(fc, 4)

        @pl.when(fc >= 4)
        def _():
            pltpu.make_async_copy(stage_v.at[pl.ds(slot * 4, 4)],
                                  out_hbm.at[0, 0], sems.at[slot]).wait()
        for k in range(8):
            stage_v[pl.ds(slot * 4, 1), pl.ds(k * 16, 16)] = accs[k]
            stage_v[pl.ds(slot * 4 + 1, 1), pl.ds(k * 16, 16)] = accs[8 + k]
            stage_v[pl.ds(slot * 4 + 2, 1), pl.ds(k * 16, 16)] = accs[16 + k]
            stage_v[pl.ds(slot * 4 + 3, 1), pl.ds(k * 16, 16)] = accs[24 + k]
        pltpu.async_copy(stage_v.at[pl.ds(slot * 4, 4)], out_hbm.at[fg, cur],
                         sems.at[slot])

    fcf = jnp.where(cur >= 0, fc + 1, fc)
    for k in range(4):
        @pl.when(fcf > k)
        def _(k=k):
            pltpu.make_async_copy(stage_v.at[pl.ds(k * 4, 4)],
                                  out_hbm.at[0, 0], sems.at[k]).wait()


def _sc_stats(Bst, srcs4, dsts, offs):
    kern = pl.kernel(
        _sc_stats_kernel,
        out_type=jax.ShapeDtypeStruct((FG, NP, 4, 128), f32),
        mesh=_mesh(),
        scratch_types=[
            pltpu.VMEM((2 * CH,), i32),
            pltpu.VMEM((2 * CH, 128), f32),
            pltpu.VMEM((2 * CH,), i32),
            pltpu.VMEM((16,), i32),
            pltpu.VMEM((16, 128), f32),
            pltpu.SemaphoreType.DMA((4,)),
            pltpu.SemaphoreType.DMA((2,)),
        ],
    )
    return kern(Bst, srcs4, dsts, offs)


def _sc_segsum_kernel(nseg, width, data_hbm, idx_hbm, keys_hbm, offs_hbm,
                      out_hbm, idx_v, rows_v, key_sm, off_sm, stage_v, sems,
                      gsem, linear):
    cid = lax.axis_index("c")
    sid = lax.axis_index("s")
    wid = sid * 2 + cid
    lo = wid * (nseg // 32)
    hi = lo + (nseg // 32)
    pltpu.sync_copy(offs_hbm.at[pl.ds(lo, 8)], off_sm.at[pl.ds(0, 8)])
    pltpu.sync_copy(offs_hbm.at[pl.ds(hi, 8)], off_sm.at[pl.ds(8, 8)])
    ovec = off_sm[pl.ds(0, 16)]
    e0 = ovec[0]
    e1 = ovec[8]
    c0 = e0 // CH
    c1 = (e1 + CH - 1) // CH
    NVR = width // 16

    init_accs = [jnp.zeros((1, 16), f32) for _ in range(NVR)]

    def edge_step(i, d, carry):
        cur = carry[0]
        fc = carry[1]
        accs = carry[2:]
        valid = jnp.logical_and(d >= lo, d < hi)
        dn = jnp.where(valid, d, -1)
        change = dn != cur

        def do_flush(ops):
            cur_, fc_, accs_ = ops[0], ops[1], ops[2:]
            slot = lax.rem(fc_, 4)

            @pl.when(cur_ >= 0)
            def _():
                @pl.when(fc_ >= 4)
                def _():
                    pltpu.make_async_copy(stage_v.at[pl.ds(slot, 1)],
                                          out_hbm.at[pl.ds(0, 1)],
                                          sems.at[slot]).wait()
                for k in range(NVR):
                    stage_v[pl.ds(slot, 1), pl.ds(k * 16, 16)] = accs_[k]
                pltpu.async_copy(stage_v.at[pl.ds(slot, 1)],
                                 out_hbm.at[pl.ds(cur_, 1)], sems.at[slot])
            newfc = jnp.where(cur_ >= 0, fc_ + 1, fc_)
            return (dn, newfc) + tuple(init_accs)

        def no_flush(ops):
            return ops

        carry2 = lax.cond(change, do_flush, no_flush, (cur, fc) + tuple(accs))
        cur2 = carry2[0]
        fc2 = carry2[1]
        accs2 = list(carry2[2:])
        for k in range(NVR):
            accs2[k] = accs2[k] + rows_v[pl.ds(i, 1), pl.ds(k * 16, 16)]
        return (cur2, fc2) + tuple(accs2)

    def group_step(g, carry):
        dvec = key_sm[pl.ds(g * 16, 16)]
        for j in range(16):
            carry = edge_step(g * 16 + j, dvec[j], carry)
        return carry

    def chunk_body(ci, carry):
        pltpu.sync_copy(keys_hbm.at[pl.ds(ci * CH, CH)], key_sm)
        if linear:
            pltpu.sync_copy(data_hbm.at[pl.ds(ci * CH, CH)], rows_v)
        else:
            pltpu.sync_copy(idx_hbm.at[pl.ds(ci * CH, CH)], idx_v)
            pltpu.async_copy(data_hbm.at[idx_v], rows_v, gsem).wait()
        return lax.fori_loop(0, CH // 16, group_step, carry)

    carry0 = (jnp.int32(-1), jnp.int32(0)) + tuple(init_accs)
    carry = lax.fori_loop(c0, c1, chunk_body, carry0)
    cur = carry[0]
    fc = carry[1]
    accs = carry[2:]

    @pl.when(cur >= 0)
    def _():
        slot = lax.rem(fc, 4)

        @pl.when(fc >= 4)
        def _():
            pltpu.make_async_copy(stage_v.at[pl.ds(slot, 1)],
                                  out_hbm.at[pl.ds(0, 1)], sems.at[slot]).wait()
        for k in range(NVR):
            stage_v[pl.ds(slot, 1), pl.ds(k * 16, 16)] = accs[k]
        pltpu.async_copy(stage_v.at[pl.ds(slot, 1)], out_hbm.at[pl.ds(cur, 1)],
                         sems.at[slot])

    fcf = jnp.where(cur >= 0, fc + 1, fc)
    for k in range(4):
        @pl.when(fcf > k)
        def _(k=k):
            pltpu.make_async_copy(stage_v.at[pl.ds(k, 1)],
                                  out_hbm.at[pl.ds(0, 1)], sems.at[k]).wait()


def _sc_edge_logsum(lxc, src_s, dst_s, offs):
    kern = pl.kernel(
        functools.partial(_sc_segsum_kernel, NP, 48, linear=False),
        out_type=jax.ShapeDtypeStruct((NP, 48), f32),
        mesh=_mesh(),
        scratch_types=[
            pltpu.VMEM((CH,), i32),
            pltpu.VMEM((CH, 128), f32),
            pltpu.VMEM((CH,), i32),
            pltpu.VMEM((16,), i32),
            pltpu.VMEM((4, 48), f32),
            pltpu.SemaphoreType.DMA((4,)),
            pltpu.SemaphoreType.DMA,
        ],
    )
    return kern(lxc, src_s, dst_s, offs)


def _sc_pool(score, batch_pad, boffs):
    kern = pl.kernel(
        functools.partial(_sc_segsum_kernel, G, 48, linear=True),
        out_type=jax.ShapeDtypeStruct((G, 48), f32),
        mesh=_mesh(),
        scratch_types=[
            pltpu.VMEM((CH,), i32),
            pltpu.VMEM((CH, 128), f32),
            pltpu.VMEM((CH,), i32),
            pltpu.VMEM((16,), i32),
            pltpu.VMEM((4, 48), f32),
            pltpu.SemaphoreType.DMA((4,)),
            pltpu.SemaphoreType.DMA,
        ],
    )
    return kern(score, score, batch_pad, boffs)


def kernel(x, edge_index, batch, pre_lin_W, pre_lin_b, pre_W, pre_b,
           post_W, post_b, lin_W, lin_b, bn_gamma, bn_beta, mlp_W, mlp_b):
    src = edge_index[0]
    dst = edge_index[1]

    perm = jnp.argsort(dst)
    src_s = src[perm].astype(i32)
    dst_s = dst[perm].astype(i32)
    offs = jnp.searchsorted(dst_s, jnp.arange(NP + 16, dtype=i32),
                            side='left').astype(i32)
    srcs4 = src_s[None, :] + (jnp.arange(FG, dtype=i32) * NP)[:, None]
    boffs = jnp.searchsorted(batch.astype(i32), jnp.arange(528, dtype=i32),
                             side='left').astype(i32)
    batch_pad = jnp.pad(batch.astype(i32), (0, NP - N), constant_values=G + 7)

    xpad = jnp.pad(x, ((0, NP - N), (0, 0)))
    olf = offs[:NP].astype(f32)[:, None]
    ohf = offs[1:NP + 1].astype(f32)[:, None]
    blf = boffs[:G].astype(f32)[:, None]
    bhf = boffs[1:G + 1].astype(f32)[:, None]

    wflat = pre_W.reshape(NL, TF, 2 * F)
    wdT = jnp.pad(wflat[:, :, :F].transpose(0, 2, 1),
                  ((0, 0), (0, 0), (0, FW - TF))).astype(bf16)
    wsT = jnp.pad(wflat[:, :, F:].transpose(0, 2, 1),
                  ((0, 0), (0, 0), (0, FW - TF))).astype(bf16)
    pbp = jnp.pad(pre_b.reshape(NL, 1, TF), ((0, 0), (0, 0), (0, FW - TF)))
    wxT = post_W[:, :, :, :F].transpose(0, 1, 3, 2).astype(bf16)
    waT = post_W[:, :, :, F:5 * F].transpose(0, 1, 3, 2).astype(bf16)
    wbT = post_W[:, :, :, 5 * F:9 * F].transpose(0, 1, 3, 2).astype(bf16)
    wcT = post_W[:, :, :, 9 * F:].transpose(0, 1, 3, 2).astype(bf16)
    pbl = post_b.reshape(NL, 1, T, FO)
    lwT = lin_W.transpose(0, 2, 1).astype(bf16)
    lbl = lin_b.reshape(NL, 1, F)

    op0, aux = _tc_prep(xpad, olf, ohf, pre_lin_W.T, pre_lin_b.reshape(1, F))

    y = op0
    bnp = jnp.zeros((2, F), f32)
    for l in range(NL):
        first = (l == 0)
        A, B0, B1, B2, B3 = _tc_ab(first, y, bnp, wdT[l], wsT[l], pbp[l])
        Bst = jnp.concatenate([B0, B1, B2, B3], axis=0)
        stats = _sc_stats(Bst, srcs4, dst_s, offs)
        y2, bn = _tc_combine(first, y, bnp, A, stats, aux,
                             wxT[l], waT[l], wbT[l], wcT[l], pbl[l],
                             lwT[l], lbl[l])
        bnp = _tc_bnparams(bn, bn_gamma[l].reshape(1, F), bn_beta[l].reshape(1, F))
        y = y2

    lxc = _tc_t1(y, bnp, xpad)
    LS = _sc_edge_logsum(lxc, src_s, dst_s, offs)
    score = _tc_t2(LS, lxc, aux)
    gsum = _sc_pool(score, batch_pad, boffs)
    out = _tc_t3(gsum, blf, bhf, mlp_W.T.astype(bf16), mlp_b.reshape(1, 1))
    return out
